# baseline (speedup 1.0000x reference)
import functools

import jax
import jax.numpy as jnp
from jax import lax
from jax.experimental import pallas as pl
from jax.experimental.pallas import tpu as pltpu
from jax.experimental.pallas import tpu_sc as plsc

_LANES = 16


def _make_lookup(B, V, D):
    info = plsc.get_sparse_core_info()
    NW = info.num_cores * info.num_subcores
    b_per_w = B // NW
    pack = 128 // D
    assert B % NW == 0 and b_per_w % _LANES == 0
    assert 128 % D == 0 and V % pack == 0

    mesh = plsc.VectorSubcoreMesh(core_axis_name="c", subcore_axis_name="s")

    @functools.partial(
        pl.kernel,
        mesh=mesh,
        out_type=jax.ShapeDtypeStruct((B * D,), jnp.float32),
        scratch_types=[
            pltpu.VMEM((b_per_w,), jnp.int32),
            pltpu.VMEM((b_per_w,), jnp.int32),
            pltpu.VMEM((b_per_w, 128), jnp.float32),
            pltpu.VMEM((b_per_w * D,), jnp.float32),
            pltpu.SemaphoreType.DMA,
        ],
        compiler_params=pltpu.CompilerParams(needs_layout_passes=False),
    )
    def lookup(values_hbm, table_hbm, out_hbm, idx_v, qidx_v, rows_v, out_v, sem):
        wid = lax.axis_index("s") * info.num_cores + lax.axis_index("c")
        base = wid * b_per_w
        pltpu.sync_copy(values_hbm.at[pl.ds(base, b_per_w)], idx_v)

        lane = lax.iota(jnp.int32, _LANES)
        pack_bits = pack.bit_length() - 1
        for h in range(b_per_w // _LANES):
            v = idx_v[pl.ds(h * _LANES, _LANES)]
            qidx_v[pl.ds(h * _LANES, _LANES)] = lax.shift_right_logical(v, pack_bits)

        pltpu.async_copy(table_hbm.at[qidx_v], rows_v, sem).wait()

        for h in range(b_per_w // _LANES):
            v = idx_v[pl.ds(h * _LANES, _LANES)]
            row = h * _LANES + lane
            col_base = (v & (pack - 1)) * D
            out_base = row * D
            for k in range(D):
                col = plsc.load_gather(rows_v, [row, col_base + k])
                plsc.store_scatter(out_v, [out_base + k], col)

        pltpu.sync_copy(out_v, out_hbm.at[pl.ds(base * D, b_per_w * D)])

    return lookup


def kernel(values, table):
    B = values.shape[0]
    V, D = table.shape
    lookup = _make_lookup(B, V, D)
    table_pk = table.reshape(V * D // 128, 128)
    out_flat = lookup(values.astype(jnp.int32), table_pk)
    return out_flat.reshape(B, D)


# speedup vs baseline: 17.5506x; 17.5506x over previous
import functools

import jax
import jax.numpy as jnp
from jax import lax
from jax.experimental import pallas as pl
from jax.experimental.pallas import tpu as pltpu
from jax.experimental.pallas import tpu_sc as plsc

_LANES = 16
_BLKW = 128


def _make_lookup(B, V, D):
    info = plsc.get_sparse_core_info()
    NW = info.num_cores * info.num_subcores
    b_per_w = B // NW
    assert B % NW == 0 and b_per_w % _LANES == 0 and D == _LANES

    mesh = plsc.VectorSubcoreMesh(core_axis_name="c", subcore_axis_name="s")

    @functools.partial(
        pl.kernel,
        mesh=mesh,
        out_type=jax.ShapeDtypeStruct((B * D,), jnp.float32),
        scratch_types=[
            pltpu.VMEM((b_per_w,), jnp.int32),
            pltpu.VMEM((b_per_w, D, _BLKW), jnp.float32),
            pltpu.VMEM((b_per_w * D,), jnp.float32),
            pltpu.SemaphoreType.DMA,
        ],
        compiler_params=pltpu.CompilerParams(
            needs_layout_passes=False, disable_bounds_checks=True
        ),
    )
    def lookup(values_hbm, tab_t_hbm, out_hbm, idx_v, blks_v, out_v, sem):
        wid = lax.axis_index("s") * info.num_cores + lax.axis_index("c")
        base = wid * b_per_w
        pltpu.sync_copy(values_hbm.at[pl.ds(base, b_per_w)], idx_v)

        lane = lax.iota(jnp.int32, _LANES)

        starts = []
        vals = []
        for j in range(b_per_w):
            vv = idx_v[pl.ds((j // _LANES) * _LANES, _LANES)]
            vj = jnp.max(jnp.where(lane == (j % _LANES), vv, 0))
            start = pl.multiple_of(
                lax.shift_left(lax.shift_right_logical(vj, 7), 7), 128
            )
            vals.append(vj)
            starts.append(start)

        copies = []
        for j in range(b_per_w):
            c = pltpu.async_copy(
                tab_t_hbm.at[:, pl.ds(starts[j], _BLKW)], blks_v.at[j], sem
            )
            copies.append(c)
        for c in copies:
            c.wait()

        for j in range(b_per_w):
            m = jnp.full((_LANES,), vals[j] - starts[j], jnp.int32)
            col = plsc.load_gather(blks_v, [jnp.full((_LANES,), j, jnp.int32), lane, m])
            plsc.store_scatter(out_v, [j * D + lane], col)

        pltpu.sync_copy(out_v, out_hbm.at[pl.ds(base * D, b_per_w * D)])

    return lookup


def kernel(values, table):
    B = values.shape[0]
    V, D = table.shape
    lookup = _make_lookup(B, V, D)
    out_flat = lookup(values.astype(jnp.int32), table.T)
    return out_flat.reshape(B, D)
